# baseline (speedup 1.0000x reference)
import functools

import jax
import jax.numpy as jnp
from jax import lax
from jax.experimental import pallas as pl
from jax.experimental.pallas import tpu as pltpu
from jax.experimental.pallas import tpu_sc as plsc

VOCAB = 100000
EMBED = 32
BATCH = 1024
CTX = 20

NUM_CORES = 2
NUM_SUBCORES = 16
NW = NUM_CORES * NUM_SUBCORES
B_PER_W = BATCH // NW
IDX_PER_W = B_PER_W * CTX
IDX_CHUNK = 128
N_CHUNKS = IDX_PER_W // IDX_CHUNK


def _pool_body(idx_hbm, table_hbm, out_hbm, idx_v, rows_v, pooled_v, sem):
    wid = lax.axis_index("s") * NUM_CORES + lax.axis_index("c")
    base = wid * IDX_PER_W
    for j in range(N_CHUNKS):
        pltpu.sync_copy(
            idx_hbm.at[pl.ds(base + j * IDX_CHUNK, IDX_CHUNK)], idx_v.at[j]
        )
    copies = [
        pltpu.async_copy(
            table_hbm.at[idx_v.at[j]],
            rows_v.at[pl.ds(j * IDX_CHUNK, IDX_CHUNK)],
            sem,
        )
        for j in range(N_CHUNKS)
    ]
    for c in copies:
        c.wait()

    def body(i, carry):
        a0 = jnp.zeros((16,), jnp.float32)
        a1 = jnp.zeros((16,), jnp.float32)
        jbase = i * CTX
        for c in range(CTX):
            a0 = a0 + rows_v[jbase + c, 0:16]
            a1 = a1 + rows_v[jbase + c, 16:32]
        pooled_v[i, 0:16] = a0
        pooled_v[i, 16:32] = a1
        return carry

    lax.fori_loop(0, B_PER_W, body, 0)
    pltpu.sync_copy(pooled_v, out_hbm.at[pl.ds(wid * B_PER_W, B_PER_W)])


def _pool_sc(idx_flat, emb_table):
    mesh = plsc.VectorSubcoreMesh(core_axis_name="c", subcore_axis_name="s")
    k = functools.partial(
        pl.kernel,
        mesh=mesh,
        out_type=jax.ShapeDtypeStruct((BATCH, EMBED), jnp.float32),
        scratch_types=[
            pltpu.VMEM((N_CHUNKS, IDX_CHUNK), jnp.int32),
            pltpu.VMEM((IDX_PER_W, 128), jnp.float32),
            pltpu.VMEM((B_PER_W, EMBED), jnp.float32),
            pltpu.SemaphoreType.DMA,
        ],
        compiler_params=pltpu.CompilerParams(use_tc_tiling_on_sc=False),
    )(_pool_body)
    return k(idx_flat, emb_table)


DT_BLK = 2048


def _detile_body(t_ref, o_ref):
    eye = jnp.eye(EMBED, dtype=jnp.float32)
    t = lax.dot_general(
        t_ref[...], eye, (((0,), (0,)), ((), ())),
        precision=lax.Precision.HIGHEST,
        preferred_element_type=jnp.float32,
    )
    o_ref[:, 0:EMBED] = t


def _detile_tc(embT):
    return pl.pallas_call(
        _detile_body,
        grid=(pl.cdiv(VOCAB, DT_BLK),),
        in_specs=[pl.BlockSpec((EMBED, DT_BLK), lambda i: (0, i))],
        out_specs=pl.BlockSpec((DT_BLK, 128), lambda i: (i, 0)),
        out_shape=jax.ShapeDtypeStruct((VOCAB, 128), jnp.float32),
    )(embT)


V_BLK = 2048


def _mm_body(p_ref, wt_ref, b_ref, o_ref):
    prod = lax.dot_general(
        wt_ref[...], p_ref[...],
        (((0,), (1,)), ((), ())),
        preferred_element_type=jnp.float32,
    )
    bias = lax.dot_general(
        b_ref[...], jnp.ones((1, BATCH), jnp.float32),
        (((0,), (0,)), ((), ())),
        preferred_element_type=jnp.float32,
    )
    o_ref[...] = prod + bias


def _project_tc(pooled, Wt, b2):
    grid = pl.cdiv(VOCAB, V_BLK)
    return pl.pallas_call(
        _mm_body,
        grid=(grid,),
        in_specs=[
            pl.BlockSpec((BATCH, EMBED), lambda i: (0, 0)),
            pl.BlockSpec((EMBED, V_BLK), lambda i: (0, i)),
            pl.BlockSpec((1, V_BLK), lambda i: (0, i)),
        ],
        out_specs=pl.BlockSpec((V_BLK, BATCH), lambda i: (i, 0)),
        out_shape=jax.ShapeDtypeStruct((VOCAB, BATCH), jnp.float32),
    )(pooled, Wt, b2)


def kernel(inputs, emb_table, W, b):
    idx_flat = inputs.astype(jnp.int32).reshape(BATCH * CTX)
    pooled = _pool_sc(idx_flat, _detile_tc(emb_table.T))
    out_t = _project_tc(pooled, W.T, b.reshape(1, VOCAB))
    return out_t.T


# speedup vs baseline: 1.0054x; 1.0054x over previous
import functools

import jax
import jax.numpy as jnp
from jax import lax
from jax.experimental import pallas as pl
from jax.experimental.pallas import tpu as pltpu
from jax.experimental.pallas import tpu_sc as plsc

VOCAB = 100000
EMBED = 32
BATCH = 1024
CTX = 20

NUM_CORES = 2
NUM_SUBCORES = 16
NW = NUM_CORES * NUM_SUBCORES
B_PER_W = BATCH // NW
IDX_PER_W = B_PER_W * CTX
IDX_CHUNK = 128
N_CHUNKS = IDX_PER_W // IDX_CHUNK


def _pool_body(idx_hbm, table_hbm, out_hbm, idx_v, rows_v, pooled_v, sem):
    wid = lax.axis_index("s") * NUM_CORES + lax.axis_index("c")
    base = wid * IDX_PER_W
    for j in range(N_CHUNKS):
        pltpu.sync_copy(
            idx_hbm.at[pl.ds(base + j * IDX_CHUNK, IDX_CHUNK)], idx_v.at[j]
        )
    copies = [
        pltpu.async_copy(
            table_hbm.at[idx_v.at[j]],
            rows_v.at[pl.ds(j * IDX_CHUNK, IDX_CHUNK)],
            sem,
        )
        for j in range(N_CHUNKS)
    ]
    for c in copies:
        c.wait()

    def body(i, carry):
        a0 = jnp.zeros((16,), jnp.float32)
        a1 = jnp.zeros((16,), jnp.float32)
        jbase = i * CTX
        for c in range(CTX):
            a0 = a0 + rows_v[jbase + c, 0:16]
            a1 = a1 + rows_v[jbase + c, 16:32]
        pooled_v[i, 0:16] = a0
        pooled_v[i, 16:32] = a1
        return carry

    lax.fori_loop(0, B_PER_W, body, 0)
    pltpu.sync_copy(pooled_v, out_hbm.at[pl.ds(wid * B_PER_W, B_PER_W)])


def _pool_sc(idx_flat, emb_table):
    mesh = plsc.VectorSubcoreMesh(core_axis_name="c", subcore_axis_name="s")
    k = functools.partial(
        pl.kernel,
        mesh=mesh,
        out_type=jax.ShapeDtypeStruct((BATCH, EMBED), jnp.float32),
        scratch_types=[
            pltpu.VMEM((N_CHUNKS, IDX_CHUNK), jnp.int32),
            pltpu.VMEM((IDX_PER_W, 128), jnp.float32),
            pltpu.VMEM((B_PER_W, EMBED), jnp.float32),
            pltpu.SemaphoreType.DMA,
        ],
        compiler_params=pltpu.CompilerParams(use_tc_tiling_on_sc=False),
    )(_pool_body)
    return k(idx_flat, emb_table)


DT_BLK = 2048


def _detile_body(t_ref, o_ref):
    eye128 = (
        lax.broadcasted_iota(jnp.int32, (EMBED, 128), 0)
        == lax.broadcasted_iota(jnp.int32, (EMBED, 128), 1)
    ).astype(jnp.float32)
    o_ref[...] = lax.dot_general(
        t_ref[...], eye128, (((0,), (0,)), ((), ())),
        precision=lax.Precision.HIGHEST,
        preferred_element_type=jnp.float32,
    )


def _detile_tc(embT):
    return pl.pallas_call(
        _detile_body,
        grid=(pl.cdiv(VOCAB, DT_BLK),),
        in_specs=[pl.BlockSpec((EMBED, DT_BLK), lambda i: (0, i))],
        out_specs=pl.BlockSpec((DT_BLK, 128), lambda i: (i, 0)),
        out_shape=jax.ShapeDtypeStruct((VOCAB, 128), jnp.float32),
    )(embT)


V_BLK = 2048


def _mm_body(p_ref, wt_ref, b_ref, o_ref):
    prod = lax.dot_general(
        wt_ref[...], p_ref[...],
        (((0,), (1,)), ((), ())),
        preferred_element_type=jnp.float32,
    )
    bias = lax.dot_general(
        b_ref[...], jnp.ones((1, BATCH), jnp.float32),
        (((0,), (0,)), ((), ())),
        preferred_element_type=jnp.float32,
    )
    o_ref[...] = prod + bias


def _project_tc(pooled, Wt, b2):
    grid = pl.cdiv(VOCAB, V_BLK)
    return pl.pallas_call(
        _mm_body,
        grid=(grid,),
        in_specs=[
            pl.BlockSpec((BATCH, EMBED), lambda i: (0, 0)),
            pl.BlockSpec((EMBED, V_BLK), lambda i: (0, i)),
            pl.BlockSpec((1, V_BLK), lambda i: (0, i)),
        ],
        out_specs=pl.BlockSpec((V_BLK, BATCH), lambda i: (i, 0)),
        out_shape=jax.ShapeDtypeStruct((VOCAB, BATCH), jnp.float32),
    )(pooled, Wt, b2)


def kernel(inputs, emb_table, W, b):
    idx_flat = inputs.astype(jnp.int32).reshape(BATCH * CTX)
    pooled = _pool_sc(idx_flat, _detile_tc(emb_table.T))
    out_t = _project_tc(pooled, W.T, b.reshape(1, VOCAB))
    return out_t.T


# speedup vs baseline: 1.0366x; 1.0310x over previous
import functools

import jax
import jax.numpy as jnp
from jax import lax
from jax.experimental import pallas as pl
from jax.experimental.pallas import tpu as pltpu
from jax.experimental.pallas import tpu_sc as plsc

VOCAB = 100000
EMBED = 32
BATCH = 1024
CTX = 20

NUM_CORES = 2
NUM_SUBCORES = 16
NW = NUM_CORES * NUM_SUBCORES
B_PER_W = BATCH // NW
IDX_PER_W = B_PER_W * CTX
IDX_CHUNK = 128
N_CHUNKS = IDX_PER_W // IDX_CHUNK


def _pool_body(idx_hbm, table_hbm, out_hbm, idx_v, rows_v, pooled_v, sem):
    wid = lax.axis_index("s") * NUM_CORES + lax.axis_index("c")
    base = wid * IDX_PER_W
    for j in range(N_CHUNKS):
        pltpu.sync_copy(
            idx_hbm.at[pl.ds(base + j * IDX_CHUNK, IDX_CHUNK)], idx_v.at[j]
        )
    copies = [
        pltpu.async_copy(
            table_hbm.at[idx_v.at[j]],
            rows_v.at[pl.ds(j * IDX_CHUNK, IDX_CHUNK)],
            sem,
        )
        for j in range(N_CHUNKS)
    ]
    for c in copies:
        c.wait()

    def body(i, carry):
        a0 = jnp.zeros((16,), jnp.float32)
        a1 = jnp.zeros((16,), jnp.float32)
        jbase = i * CTX
        for c in range(CTX):
            a0 = a0 + rows_v[jbase + c, 0:16]
            a1 = a1 + rows_v[jbase + c, 16:32]
        pooled_v[i, 0:16] = a0
        pooled_v[i, 16:32] = a1
        return carry

    lax.fori_loop(0, B_PER_W, body, 0)
    pltpu.sync_copy(pooled_v, out_hbm.at[pl.ds(wid * B_PER_W, B_PER_W)])


def _pool_sc(idx_flat, emb_table):
    mesh = plsc.VectorSubcoreMesh(core_axis_name="c", subcore_axis_name="s")
    k = functools.partial(
        pl.kernel,
        mesh=mesh,
        out_type=jax.ShapeDtypeStruct((BATCH, EMBED), jnp.float32),
        scratch_types=[
            pltpu.VMEM((N_CHUNKS, IDX_CHUNK), jnp.int32),
            pltpu.VMEM((IDX_PER_W, 128), jnp.float32),
            pltpu.VMEM((B_PER_W, EMBED), jnp.float32),
            pltpu.SemaphoreType.DMA,
        ],
        compiler_params=pltpu.CompilerParams(use_tc_tiling_on_sc=False),
    )(_pool_body)
    return k(idx_flat, emb_table)


DT_BLK = 16384


def _detile_body(t_ref, o_ref):
    eye128 = (
        lax.broadcasted_iota(jnp.int32, (EMBED, 128), 0)
        == lax.broadcasted_iota(jnp.int32, (EMBED, 128), 1)
    ).astype(jnp.float32)
    o_ref[...] = lax.dot_general(
        t_ref[...], eye128, (((0,), (0,)), ((), ())),
        precision=lax.Precision.HIGHEST,
        preferred_element_type=jnp.float32,
    )


def _detile_tc(embT):
    return pl.pallas_call(
        _detile_body,
        grid=(pl.cdiv(VOCAB, DT_BLK),),
        in_specs=[pl.BlockSpec((EMBED, DT_BLK), lambda i: (0, i))],
        out_specs=pl.BlockSpec((DT_BLK, 128), lambda i: (i, 0)),
        out_shape=jax.ShapeDtypeStruct((VOCAB, 128), jnp.float32),
    )(embT)


V_BLK = 2048


def _mm_body(p_ref, wt_ref, b_ref, o_ref):
    prod = lax.dot_general(
        wt_ref[...], p_ref[...],
        (((0,), (1,)), ((), ())),
        preferred_element_type=jnp.float32,
    )
    bias = lax.dot_general(
        b_ref[...], jnp.ones((1, BATCH), jnp.float32),
        (((0,), (0,)), ((), ())),
        preferred_element_type=jnp.float32,
    )
    o_ref[...] = prod + bias


def _project_tc(pooled, Wt, b2):
    grid = pl.cdiv(VOCAB, V_BLK)
    return pl.pallas_call(
        _mm_body,
        grid=(grid,),
        in_specs=[
            pl.BlockSpec((BATCH, EMBED), lambda i: (0, 0)),
            pl.BlockSpec((EMBED, V_BLK), lambda i: (0, i)),
            pl.BlockSpec((1, V_BLK), lambda i: (0, i)),
        ],
        out_specs=pl.BlockSpec((V_BLK, BATCH), lambda i: (i, 0)),
        out_shape=jax.ShapeDtypeStruct((VOCAB, BATCH), jnp.float32),
    )(pooled, Wt, b2)


def kernel(inputs, emb_table, W, b):
    idx_flat = inputs.astype(jnp.int32).reshape(BATCH * CTX)
    pooled = _pool_sc(idx_flat, _detile_tc(emb_table.T))
    out_t = _project_tc(pooled, W.T, b.reshape(1, VOCAB))
    return out_t.T


# speedup vs baseline: 1.1277x; 1.0879x over previous
import functools

import jax
import jax.numpy as jnp
from jax import lax
from jax.experimental import pallas as pl
from jax.experimental.pallas import tpu as pltpu
from jax.experimental.pallas import tpu_sc as plsc

VOCAB = 100000
EMBED = 32
BATCH = 1024
CTX = 20

NUM_CORES = 2
NUM_SUBCORES = 16
NW = NUM_CORES * NUM_SUBCORES
B_PER_W = BATCH // NW
IDX_PER_W = B_PER_W * CTX
IDX_CHUNK = 128
N_CHUNKS = IDX_PER_W // IDX_CHUNK


def _pool_body(idx_hbm, table_hbm, out_hbm, idx_v, rows_v, pooled_v, sem):
    wid = lax.axis_index("s") * NUM_CORES + lax.axis_index("c")
    base = wid * IDX_PER_W
    for j in range(N_CHUNKS):
        pltpu.sync_copy(
            idx_hbm.at[pl.ds(base + j * IDX_CHUNK, IDX_CHUNK)], idx_v.at[j]
        )
    copies = [
        pltpu.async_copy(
            table_hbm.at[idx_v.at[j]],
            rows_v.at[pl.ds(j * IDX_CHUNK, IDX_CHUNK)],
            sem,
        )
        for j in range(N_CHUNKS)
    ]
    for c in copies:
        c.wait()

    def body(i, carry):
        a0 = jnp.zeros((16,), jnp.float32)
        a1 = jnp.zeros((16,), jnp.float32)
        jbase = i * CTX
        for c in range(CTX):
            a0 = a0 + rows_v[jbase + c, 0:16]
            a1 = a1 + rows_v[jbase + c, 16:32]
        pooled_v[i, 0:16] = a0
        pooled_v[i, 16:32] = a1
        return carry

    lax.fori_loop(0, B_PER_W, body, 0)
    pltpu.sync_copy(pooled_v, out_hbm.at[pl.ds(wid * B_PER_W, B_PER_W)])


def _pool_sc(idx_flat, emb_table):
    mesh = plsc.VectorSubcoreMesh(core_axis_name="c", subcore_axis_name="s")
    k = functools.partial(
        pl.kernel,
        mesh=mesh,
        out_type=jax.ShapeDtypeStruct((BATCH, EMBED), jnp.float32),
        scratch_types=[
            pltpu.VMEM((N_CHUNKS, IDX_CHUNK), jnp.int32),
            pltpu.VMEM((IDX_PER_W, 128), jnp.float32),
            pltpu.VMEM((B_PER_W, EMBED), jnp.float32),
            pltpu.SemaphoreType.DMA,
        ],
        compiler_params=pltpu.CompilerParams(use_tc_tiling_on_sc=False),
    )(_pool_body)
    return k(idx_flat, emb_table)


DT_BLK = 16384


def _detile_body(t_ref, o_ref):
    eye128 = (
        lax.broadcasted_iota(jnp.int32, (EMBED, 128), 0)
        == lax.broadcasted_iota(jnp.int32, (EMBED, 128), 1)
    ).astype(jnp.float32)
    x = t_ref[...]
    x_hi = x.astype(jnp.bfloat16).astype(jnp.float32)
    x_lo = x - x_hi
    dims = (((0,), (0,)), ((), ()))
    o_ref[...] = lax.dot_general(
        x_hi, eye128, dims, preferred_element_type=jnp.float32
    ) + lax.dot_general(
        x_lo, eye128, dims, preferred_element_type=jnp.float32
    )


def _detile_tc(embT):
    return pl.pallas_call(
        _detile_body,
        grid=(pl.cdiv(VOCAB, DT_BLK),),
        in_specs=[pl.BlockSpec((EMBED, DT_BLK), lambda i: (0, i))],
        out_specs=pl.BlockSpec((DT_BLK, 128), lambda i: (i, 0)),
        out_shape=jax.ShapeDtypeStruct((VOCAB, 128), jnp.float32),
    )(embT)


V_BLK = 2048


def _mm_body(p_ref, wt_ref, b_ref, o_ref):
    prod = lax.dot_general(
        wt_ref[...], p_ref[...],
        (((0,), (1,)), ((), ())),
        preferred_element_type=jnp.float32,
    )
    bias = lax.dot_general(
        b_ref[...], jnp.ones((1, BATCH), jnp.float32),
        (((0,), (0,)), ((), ())),
        preferred_element_type=jnp.float32,
    )
    o_ref[...] = prod + bias


def _project_tc(pooled, Wt, b2):
    grid = pl.cdiv(VOCAB, V_BLK)
    return pl.pallas_call(
        _mm_body,
        grid=(grid,),
        in_specs=[
            pl.BlockSpec((BATCH, EMBED), lambda i: (0, 0)),
            pl.BlockSpec((EMBED, V_BLK), lambda i: (0, i)),
            pl.BlockSpec((1, V_BLK), lambda i: (0, i)),
        ],
        out_specs=pl.BlockSpec((V_BLK, BATCH), lambda i: (i, 0)),
        out_shape=jax.ShapeDtypeStruct((VOCAB, BATCH), jnp.float32),
    )(pooled, Wt, b2)


def kernel(inputs, emb_table, W, b):
    idx_flat = inputs.astype(jnp.int32).reshape(BATCH * CTX)
    pooled = _pool_sc(idx_flat, _detile_tc(emb_table.T))
    out_t = _project_tc(pooled, W.T, b.reshape(1, VOCAB))
    return out_t.T
